# baseline (speedup 1.0000x reference)
import functools

import jax
import jax.numpy as jnp
from jax import lax
from jax.experimental import pallas as pl
from jax.experimental.pallas import tpu as pltpu
from jax.experimental.pallas import tpu_sc as plsc

HEAD_HALF = 64
NC = 2
NS = 16
NW = NC * NS
IDX_CHUNK = 128
PHASE = 256
CPP = PHASE // IDX_CHUNK


def _make_gather(total_rows: int):
    assert total_rows % (NW * PHASE) == 0
    b_per_w = total_rows // NW
    n_chunks = b_per_w // IDX_CHUNK
    n_phases = b_per_w // PHASE
    mesh = plsc.VectorSubcoreMesh(core_axis_name="c", subcore_axis_name="s")

    out_sds = jax.ShapeDtypeStruct((total_rows, HEAD_HALF), jnp.float32)
    row_buf = pltpu.VMEM((PHASE, HEAD_HALF), jnp.float32)

    @functools.partial(
        pl.kernel,
        mesh=mesh,
        out_type=(out_sds, out_sds),
        compiler_params=pltpu.CompilerParams(use_tc_tiling_on_sc=False),
        scratch_types=[
            pltpu.VMEM((n_chunks, IDX_CHUNK), jnp.int32),
            row_buf, row_buf, row_buf, row_buf,
            pltpu.SemaphoreType.DMA,
            pltpu.SemaphoreType.DMA,
        ],
    )
    def gather(idx_hbm, cos_hbm, sin_hbm, cos_out, sin_out,
               idx_v, rows_c0, rows_c1, rows_s0, rows_s1, sem_g, sem_w):
        wid = lax.axis_index("s") * NC + lax.axis_index("c")
        base = wid * b_per_w
        pltpu.sync_copy(idx_hbm.at[wid], idx_v)
        bufs_c = (rows_c0, rows_c1)
        bufs_s = (rows_s0, rows_s1)
        gather_waits = [None] * n_phases
        wb_waits = [None] * n_phases
        for p in range(n_phases):
            bc, bs = bufs_c[p % 2], bufs_s[p % 2]
            if p >= 2:
                for w in wb_waits[p - 2]:
                    w.wait()
            gw = []
            for j in range(CPP):
                chunk = p * CPP + j
                dst = pl.ds(j * IDX_CHUNK, IDX_CHUNK)
                gw.append(pltpu.async_copy(
                    cos_hbm.at[idx_v.at[chunk]], bc.at[dst], sem_g))
                gw.append(pltpu.async_copy(
                    sin_hbm.at[idx_v.at[chunk]], bs.at[dst], sem_g))
            gather_waits[p] = gw
            if p >= 1:
                q = p - 1
                for w in gather_waits[q]:
                    w.wait()
                out_sl = pl.ds(base + q * PHASE, PHASE)
                wb_waits[q] = [
                    pltpu.async_copy(bufs_c[q % 2], cos_out.at[out_sl], sem_w),
                    pltpu.async_copy(bufs_s[q % 2], sin_out.at[out_sl], sem_w),
                ]
        q = n_phases - 1
        for w in gather_waits[q]:
            w.wait()
        out_sl = pl.ds(base + q * PHASE, PHASE)
        wb_waits[q] = [
            pltpu.async_copy(bufs_c[q % 2], cos_out.at[out_sl], sem_w),
            pltpu.async_copy(bufs_s[q % 2], sin_out.at[out_sl], sem_w),
        ]
        for q in (n_phases - 2, n_phases - 1):
            for w in wb_waits[q]:
                w.wait()

    return gather


def kernel(position_ids, cos_cache, sin_cache):
    batch, seq = position_ids.shape
    total = batch * seq
    idx = position_ids.reshape(NW, total // NW // IDX_CHUNK, IDX_CHUNK)
    cos_flat, sin_flat = _make_gather(total)(idx, cos_cache, sin_cache)
    shape = (batch, seq, HEAD_HALF)
    return cos_flat.reshape(shape), sin_flat.reshape(shape)
